# baseline (speedup 1.0000x reference)
import jax
import jax.numpy as jnp
from jax import lax
from jax.experimental import pallas as pl
from jax.experimental.pallas import tpu as pltpu
from jax.experimental.pallas import tpu_sc as plsc

B = 4
N = 4096
J = 4096

J_TC = 2048
J_SC = J - J_TC

NC = 2
NS = 16
NW = NC * NS
W = J_SC // NW
L = 16
G = W // L
R = 512
NCHUNK = N // R

CT = 512


def _argmax_sc_body(x_hbm, out_hbm, buf, outv, sem):
    cid = lax.axis_index("c")
    sid = lax.axis_index("s")
    wid = sid * NC + cid
    j0 = J_TC + wid * W

    neg_inf = jnp.full((L,), -jnp.inf, dtype=jnp.float32)
    zero_i = jnp.zeros((L,), dtype=jnp.int32)

    for b in range(B):
        pltpu.async_copy(
            x_hbm.at[b, pl.ds(0, R), pl.ds(j0, W)], buf.at[0], sem.at[0])
        pltpu.async_copy(
            x_hbm.at[b, pl.ds(R, R), pl.ds(j0, W)], buf.at[1], sem.at[1])

        init = ([neg_inf] * G, [zero_i] * G)

        def pair_body(p, carry, b=b):
            ms, idxs = carry
            for k in range(2):
                t = 2 * p + k
                pltpu.make_async_copy(
                    x_hbm.at[b, pl.ds(0, R), pl.ds(j0, W)],
                    buf.at[k], sem.at[k]).wait()
                base = t * R

                def row_body(i, c2, k=k, base=base):
                    ms2, idxs2 = c2
                    rowidx = jnp.full((L,), 0, jnp.int32) + (base + i)
                    nms, nidxs = [], []
                    for g in range(G):
                        v = buf[k, i, pl.ds(g * L, L)]
                        cond = v > ms2[g]
                        nms.append(jnp.maximum(v, ms2[g]))
                        nidxs.append(jnp.where(cond, rowidx, idxs2[g]))
                    return nms, nidxs

                ms, idxs = lax.fori_loop(0, R, row_body, (ms, idxs),
                                         unroll=8)

                @pl.when(t + 2 < NCHUNK)
                def _(k=k, t=t, b=b):
                    pltpu.async_copy(
                        x_hbm.at[b, pl.ds((t + 2) * R, R), pl.ds(j0, W)],
                        buf.at[k], sem.at[k])

            return ms, idxs

        ms, idxs = lax.fori_loop(0, NCHUNK // 2, pair_body, init)
        for g in range(G):
            outv[pl.ds(g * L, L)] = idxs[g]
        pltpu.sync_copy(outv, out_hbm.at[b, pl.ds(wid * W, W)])


def _argmax_tc_body(x_ref, o_ref):
    m0 = jnp.full((8, CT), -jnp.inf, dtype=jnp.float32)
    i0 = lax.broadcasted_iota(jnp.int32, (8, CT), 0)

    def step(r, carry):
        m, idx, rid = carry
        v = x_ref[0, pl.ds(r * 8, 8), :]
        cond = v > m
        m2 = jnp.maximum(v, m)
        idx2 = jnp.where(cond, rid, idx)
        return m2, idx2, rid + 8

    m, idx, _ = lax.fori_loop(0, N // 8, step, (m0, i0, i0), unroll=8)

    mm, ii = m, idx
    for half in (4, 2, 1):
        ma, ia = mm[:half], ii[:half]
        mb, ib = mm[half:2 * half], ii[half:2 * half]
        take_b = (mb > ma) | ((mb == ma) & (ib < ia))
        mm = jnp.where(take_b, mb, ma)
        ii = jnp.where(take_b, ib, ia)
    o_ref[0, 0, :] = ii[0]


@jax.jit
def _argmax_hybrid(x):
    mesh = plsc.VectorSubcoreMesh(
        core_axis_name="c", subcore_axis_name="s",
        num_cores=NC, num_subcores=NS)
    out_sc = pl.kernel(
        _argmax_sc_body,
        out_type=jax.ShapeDtypeStruct((B, J_SC), jnp.int32),
        mesh=mesh,
        compiler_params=pltpu.CompilerParams(use_tc_tiling_on_sc=False),
        scratch_types=[
            pltpu.VMEM((2, R, W), jnp.float32),
            pltpu.VMEM((W,), jnp.int32),
            pltpu.SemaphoreType.DMA((2,)),
        ],
    )(x)

    out_tc = pl.pallas_call(
        _argmax_tc_body,
        grid=(B, J_TC // CT),
        in_specs=[pl.BlockSpec((1, N, CT), lambda b, c: (b, 0, c))],
        out_specs=pl.BlockSpec((1, 1, CT), lambda b, c: (b, 0, c)),
        out_shape=jax.ShapeDtypeStruct((B, 1, J_TC), jnp.int32),
    )(x)

    return jnp.concatenate([out_tc.reshape(B, J_TC), out_sc], axis=1)


def kernel(x):
    return _argmax_hybrid(x)


# speedup vs baseline: 2.8598x; 2.8598x over previous
import jax
import jax.numpy as jnp
from jax import lax
from jax.experimental import pallas as pl
from jax.experimental.pallas import tpu as pltpu
from jax.experimental.pallas import tpu_sc as plsc

B = 4
N = 4096
J = 4096

RS = 2560
NROWS_SC = N - RS

NC = 2
NS = 16
NW = NC * NS
W = J // NW
L = 16
G = W // L
R = 256
NCHUNK = NROWS_SC // R

CT = 512


def _argmax_sc_body(x_hbm, outm_hbm, outi_hbm, buf, outmv, outiv, sem):
    cid = lax.axis_index("c")
    sid = lax.axis_index("s")
    wid = sid * NC + cid
    j0 = wid * W

    neg_inf = jnp.full((L,), -jnp.inf, dtype=jnp.float32)
    zero_i = jnp.zeros((L,), dtype=jnp.int32)

    for b in range(B):
        pltpu.async_copy(
            x_hbm.at[b, pl.ds(RS, R), pl.ds(j0, W)], buf.at[0], sem.at[0])
        pltpu.async_copy(
            x_hbm.at[b, pl.ds(RS + R, R), pl.ds(j0, W)], buf.at[1],
            sem.at[1])

        init = ([neg_inf] * G, [zero_i] * G)

        def pair_body(p, carry, b=b):
            ms, idxs = carry
            for k in range(2):
                t = 2 * p + k
                pltpu.make_async_copy(
                    x_hbm.at[b, pl.ds(RS, R), pl.ds(j0, W)],
                    buf.at[k], sem.at[k]).wait()
                base = RS + t * R

                def row_body(i, c2, k=k, base=base):
                    ms2, idxs2 = c2
                    rowidx = jnp.full((L,), 0, jnp.int32) + (base + i)
                    nms, nidxs = [], []
                    for g in range(G):
                        v = buf[k, i, pl.ds(g * L, L)]
                        cond = v > ms2[g]
                        nms.append(jnp.maximum(v, ms2[g]))
                        nidxs.append(jnp.where(cond, rowidx, idxs2[g]))
                    return nms, nidxs

                ms, idxs = lax.fori_loop(0, R, row_body, (ms, idxs),
                                         unroll=8)

                @pl.when(t + 2 < NCHUNK)
                def _(k=k, t=t, b=b):
                    pltpu.async_copy(
                        x_hbm.at[b, pl.ds(RS + (t + 2) * R, R),
                                 pl.ds(j0, W)],
                        buf.at[k], sem.at[k])

            return ms, idxs

        ms, idxs = lax.fori_loop(0, NCHUNK // 2, pair_body, init)
        for g in range(G):
            outmv[pl.ds(g * L, L)] = ms[g]
            outiv[pl.ds(g * L, L)] = idxs[g]
        pltpu.sync_copy(outmv, outm_hbm.at[b, pl.ds(j0, W)])
        pltpu.sync_copy(outiv, outi_hbm.at[b, pl.ds(j0, W)])


def _argmax_tc_body(x_ref, om_ref, oi_ref):
    m0 = jnp.full((8, CT), -jnp.inf, dtype=jnp.float32)
    i0 = lax.broadcasted_iota(jnp.int32, (8, CT), 0)

    def step(r, carry):
        m, idx, rid = carry
        v = x_ref[0, pl.ds(r * 8, 8), :]
        cond = v > m
        m2 = jnp.maximum(v, m)
        idx2 = jnp.where(cond, rid, idx)
        return m2, idx2, rid + 8

    m, idx, _ = lax.fori_loop(0, RS // 8, step, (m0, i0, i0), unroll=8)

    mm, ii = m, idx
    for half in (4, 2, 1):
        ma, ia = mm[:half], ii[:half]
        mb, ib = mm[half:2 * half], ii[half:2 * half]
        take_b = (mb > ma) | ((mb == ma) & (ib < ia))
        mm = jnp.where(take_b, mb, ma)
        ii = jnp.where(take_b, ib, ia)
    om_ref[0, 0, :] = mm[0]
    oi_ref[0, 0, :] = ii[0]


@jax.jit
def _argmax_hybrid(x):
    mesh = plsc.VectorSubcoreMesh(
        core_axis_name="c", subcore_axis_name="s",
        num_cores=NC, num_subcores=NS)
    outm_sc, outi_sc = pl.kernel(
        _argmax_sc_body,
        out_type=(jax.ShapeDtypeStruct((B, J), jnp.float32),
                  jax.ShapeDtypeStruct((B, J), jnp.int32)),
        mesh=mesh,
        scratch_types=[
            pltpu.VMEM((2, R, W), jnp.float32),
            pltpu.VMEM((W,), jnp.float32),
            pltpu.VMEM((W,), jnp.int32),
            pltpu.SemaphoreType.DMA((2,)),
        ],
    )(x)

    outm_tc, outi_tc = pl.pallas_call(
        _argmax_tc_body,
        grid=(B, J // CT),
        in_specs=[pl.BlockSpec((1, RS, CT), lambda b, c: (b, 0, c))],
        out_specs=[pl.BlockSpec((1, 1, CT), lambda b, c: (b, 0, c)),
                   pl.BlockSpec((1, 1, CT), lambda b, c: (b, 0, c))],
        out_shape=[jax.ShapeDtypeStruct((B, 1, J), jnp.float32),
                   jax.ShapeDtypeStruct((B, 1, J), jnp.int32)],
    )(x)

    outm_tc = outm_tc.reshape(B, J)
    outi_tc = outi_tc.reshape(B, J)
    return jnp.where(outm_sc > outm_tc, outi_sc, outi_tc)


def kernel(x):
    return _argmax_hybrid(x)
